# Initial kernel scaffold; baseline (speedup 1.0000x reference)
import jax
import jax.numpy as jnp
from jax.experimental import pallas as pl


def kernel(inputs, embedding):
    raise NotImplementedError("write your pallas kernel here")


# speedup vs baseline: 1.5752x; 1.5752x over previous
import functools

import jax
import jax.numpy as jnp
from jax import lax
from jax.experimental import pallas as pl
from jax.experimental.pallas import tpu as pltpu
from jax.experimental.pallas import tpu_sc as plsc

_BATCH = 16384
_FIELDS = 26
_FEAT = 32
_BF = _BATCH * _FIELDS
_NW = 32
_ROWS_W = _BF // _NW
_CHUNK = 1024
_NCHUNK = _ROWS_W // _CHUNK


def _embed_body(idx_hbm, table_hbm, out_hbm, idx_v, buf0, buf1, sem0, sem1):
    c = lax.axis_index("c")
    s = lax.axis_index("s")
    wid = s * 2 + c
    base = wid * _ROWS_W
    pltpu.sync_copy(idx_hbm.at[pl.ds(base, _ROWS_W)], idx_v)

    bufs = (buf0, buf1)
    sems = (sem0, sem1)

    def start(j):
        return pltpu.async_copy(
            table_hbm.at[idx_v.at[pl.ds(j * _CHUNK, _CHUNK)]],
            bufs[j % 2], sems[j % 2])

    descs = [None] * _NCHUNK
    descs[0] = start(0)
    for j in range(_NCHUNK):
        if j + 1 < _NCHUNK:
            descs[j + 1] = start(j + 1)
        descs[j].wait()
        pltpu.sync_copy(bufs[j % 2],
                        out_hbm.at[pl.ds(base + j * _CHUNK, _CHUNK)])


_embed_call = functools.partial(
    pl.kernel,
    out_type=jax.ShapeDtypeStruct((_BF, _FEAT), jnp.float32),
    mesh=plsc.VectorSubcoreMesh(core_axis_name="c", subcore_axis_name="s"),
    scratch_types=[
        pltpu.VMEM((_ROWS_W,), jnp.int32),
        pltpu.VMEM((_CHUNK, _FEAT), jnp.float32),
        pltpu.VMEM((_CHUNK, _FEAT), jnp.float32),
        pltpu.SemaphoreType.DMA,
        pltpu.SemaphoreType.DMA,
    ],
    compiler_params=pltpu.CompilerParams(use_tc_tiling_on_sc=False),
)(_embed_body)


def kernel(inputs, embedding):
    idx = inputs.reshape(_BF).astype(jnp.int32)
    out = _embed_call(idx, embedding)
    return out.reshape(_BATCH, _FIELDS, _FEAT)
